# baseline (speedup 1.0000x reference)
import functools

import jax
import jax.numpy as jnp
from jax import lax
from jax.experimental import pallas as pl
from jax.experimental.pallas import tpu as pltpu
from jax.experimental.pallas import tpu_sc as plsc

N = 10000
E = 320000
D_IN = 128
D_OUT = 512

NC = 2
NS = 16
NW = NC * NS
NPAD = 10240
RPS = NPAD // NS
EPW = E // NW
K = 80
NCHUNK = EPW // K

EPS = E // NS
NCHUNK2 = EPS // K
NHALF = 5120
NACC = 5248
ZPS = NACC // NS
OPS = NHALF // NS
K2 = 128
C = 20096
CCH = C // K2


def _degree_body(src_hbm, dst_hbm, zeros_hbm, deg_hbm, bsrc_hbm, bdst_hbm,
                 cnt_hbm, srcv, dstv, hist, tmp, bsrcv, bdstv, cntv, hist_sh):
    cid = lax.axis_index("c")
    sid = lax.axis_index("s")

    pltpu.sync_copy(zeros_hbm, hist)
    pltpu.sync_copy(src_hbm.at[sid], srcv)
    pltpu.sync_copy(dst_hbm.at[sid], dstv)

    ones16 = jnp.ones((16,), jnp.float32)

    @pl.loop(cid * (NCHUNK2 // 2), (cid + 1) * (NCHUNK2 // 2))
    def _chunk(j):
        for l in range(K // 16):
            idx = dstv[j, pl.ds(l * 16, 16)]
            plsc.addupdate_scatter(hist, [idx], ones16)

    lanes = lax.iota(jnp.int32, 16)

    @pl.loop(0, C // 16)
    def _pf(i):
        bdstv[pl.ds(i * 16, 16)] = NHALF + lanes
        bsrcv[pl.ds(i * 16, 16)] = jnp.zeros((16,), jnp.int32)

    base = cid * NHALF

    @pl.loop(0, NCHUNK2, init_carry=jnp.int32(0))
    def _part(j, off):
        for l in range(K // 16):
            g = pl.ds(l * 16, 16)
            v = dstv[j, g] - base
            ok = (v >= 0) & (v < NHALF)
            plsc.store_compressed(bdstv.at[pl.ds(off, 16)], v, mask=ok)
            plsc.store_compressed(bsrcv.at[pl.ds(off, 16)], srcv[j, g],
                                  mask=ok)
            off = off + jnp.sum(ok.astype(jnp.int32))
        return off

    off = _part
    pltpu.sync_copy(bsrcv, bsrc_hbm.at[cid, sid])
    pltpu.sync_copy(bdstv, bdst_hbm.at[cid, sid])
    cntv[...] = jnp.full((16,), 1, jnp.int32) * off
    pltpu.sync_copy(cntv, cnt_hbm.at[cid, sid])

    pltpu.sync_copy(hist, hist_sh.at[sid])
    plsc.subcore_barrier()

    cols = pl.ds(sid * RPS, RPS)
    pltpu.sync_copy(hist_sh.at[0, cols], hist.at[pl.ds(0, RPS)])
    acc = hist.at[pl.ds(0, RPS)]
    for t in range(1, NS):
        pltpu.sync_copy(hist_sh.at[t, cols], tmp)
        for i in range(RPS // 16):
            g = pl.ds(i * 16, 16)
            acc[g] = acc[g] + tmp[g]

    pltpu.sync_copy(acc, deg_hbm.at[cid, cols])


def _agg_body(y_hbm, src_hbm, dst_hbm, cnt_hbm, agg_hbm, srcv, dstv, cntv,
              rows, sem0, sem1, ssem0, ssem1, agg_sh):
    cid = lax.axis_index("c")
    sid = lax.axis_index("s")

    @pl.loop(0, K2)
    def _z(i):
        for j in range(D_IN // 16):
            rows[i, pl.ds(j * 16, 16)] = jnp.zeros((16,), dtype=jnp.float32)

    buf0 = rows.at[pl.ds(0, K2)]
    buf1 = rows.at[pl.ds(K2, K2)]
    for c in range(ZPS // K2):
        pltpu.sync_copy(buf0, agg_sh.at[pl.ds(sid * ZPS + c * K2, K2)])
    pltpu.sync_copy(
        buf0.at[pl.ds(0, ZPS % K2)],
        agg_sh.at[pl.ds(sid * ZPS + (ZPS // K2) * K2, ZPS % K2)],
    )
    plsc.subcore_barrier()

    pltpu.sync_copy(src_hbm.at[cid, sid], srcv)
    pltpu.sync_copy(dst_hbm.at[cid, sid], dstv)
    pltpu.sync_copy(cnt_hbm.at[cid, sid], cntv)

    count = jnp.max(cntv[...])
    m2 = jnp.maximum(2, 2 * ((count + 2 * K2 - 1) // (2 * K2)))

    def _gather(j, buf, sem):
        return pltpu.async_copy(y_hbm.at[srcv.at[j]], buf, sem)

    def _gwait(j, buf, sem):
        pltpu.make_async_copy(y_hbm.at[srcv.at[j]], buf, sem).wait()

    def _scatter(j, buf, sem):
        return pltpu.async_copy(buf, agg_sh.at[dstv.at[j]], sem, add=True)

    def _swait(j, buf, sem):
        pltpu.make_async_copy(buf, agg_sh.at[dstv.at[j]], sem).wait()

    _gather(0, buf0, sem0)
    _gwait(0, buf0, sem0)
    _scatter(0, buf0, ssem0)
    _gather(1, buf1, sem1)

    @pl.loop(1, m2 // 2)
    def _chunk(g):
        j = 2 * g
        _swait(j - 2, buf0, ssem0)
        _gather(j, buf0, sem0)
        _gwait(j - 1, buf1, sem1)
        _scatter(j - 1, buf1, ssem1)
        _swait(j - 1, buf1, ssem1)
        _gather(j + 1, buf1, sem1)
        _gwait(j, buf0, sem0)
        _scatter(j, buf0, ssem0)

    _gwait(m2 - 1, buf1, sem1)
    _scatter(m2 - 1, buf1, ssem1)
    _swait(m2 - 2, buf0, ssem0)
    _swait(m2 - 1, buf1, ssem1)

    plsc.subcore_barrier()
    pltpu.sync_copy(
        agg_sh.at[pl.ds(sid * OPS, OPS)],
        agg_hbm.at[cid, pl.ds(sid * OPS, OPS)],
    )


@functools.cache
def _sc_kernels():
    mesh = plsc.VectorSubcoreMesh(
        core_axis_name="c", subcore_axis_name="s", num_cores=NC, num_subcores=NS
    )
    sc_degree = functools.partial(
        pl.kernel,
        out_type=(
            jax.ShapeDtypeStruct((NC, NPAD), jnp.float32),
            jax.ShapeDtypeStruct((NC, NS, C), jnp.int32),
            jax.ShapeDtypeStruct((NC, NS, C), jnp.int32),
            jax.ShapeDtypeStruct((NC, NS, 16), jnp.int32),
        ),
        mesh=mesh,
        scratch_types=[
            pltpu.VMEM((NCHUNK2, K), jnp.int32),
            pltpu.VMEM((NCHUNK2, K), jnp.int32),
            pltpu.VMEM((NPAD,), jnp.float32),
            pltpu.VMEM((RPS,), jnp.float32),
            pltpu.VMEM((C,), jnp.int32),
            pltpu.VMEM((C,), jnp.int32),
            pltpu.VMEM((16,), jnp.int32),
            pltpu.VMEM_SHARED((NS, NPAD), jnp.float32),
        ],
        compiler_params=pltpu.CompilerParams(needs_layout_passes=False),
    )(_degree_body)
    sc_aggregate = functools.partial(
        pl.kernel,
        out_type=jax.ShapeDtypeStruct((NC, NHALF, D_IN), jnp.float32),
        mesh=mesh,
        scratch_types=[
            pltpu.VMEM((CCH, K2), jnp.int32),
            pltpu.VMEM((CCH, K2), jnp.int32),
            pltpu.VMEM((16,), jnp.int32),
            pltpu.VMEM((2 * K2, D_IN), jnp.float32),
            pltpu.SemaphoreType.DMA,
            pltpu.SemaphoreType.DMA,
            pltpu.SemaphoreType.DMA,
            pltpu.SemaphoreType.DMA,
            pltpu.VMEM_SHARED((NACC, D_IN), jnp.float32),
        ],
        compiler_params=pltpu.CompilerParams(needs_layout_passes=False),
    )(_agg_body)
    return sc_degree, sc_aggregate


_ROWS_B = 400


def _scale_body(x_ref, d0_ref, d1_ref, y_ref, s_ref):
    s = lax.rsqrt(d0_ref[...] + d1_ref[...] + 1.0)
    y_ref[...] = x_ref[...] * s
    s_ref[...] = s


_tc_scale = pl.pallas_call(
    _scale_body,
    grid=(N // _ROWS_B,),
    in_specs=[pl.BlockSpec((_ROWS_B, D_IN), lambda i: (i, 0))] * 3,
    out_specs=[pl.BlockSpec((_ROWS_B, D_IN), lambda i: (i, 0))] * 2,
    out_shape=[jax.ShapeDtypeStruct((N, D_IN), jnp.float32)] * 2,
)


def _final_body(a_ref, y_ref, s_ref, w_ref, b_ref, o_ref):
    z = (a_ref[...] + y_ref[...]) * s_ref[...]
    acc = jnp.dot(z, w_ref[...], preferred_element_type=jnp.float32)
    o_ref[...] = jnp.maximum(acc + b_ref[...][0:1, :], 0.0)


_tc_final = pl.pallas_call(
    _final_body,
    grid=(N // _ROWS_B,),
    in_specs=[
        pl.BlockSpec((_ROWS_B, D_IN), lambda i: (i, 0)),
        pl.BlockSpec((_ROWS_B, D_IN), lambda i: (i, 0)),
        pl.BlockSpec((_ROWS_B, D_IN), lambda i: (i, 0)),
        pl.BlockSpec((D_IN, D_OUT), lambda i: (0, 0)),
        pl.BlockSpec((8, D_OUT), lambda i: (0, 0)),
    ],
    out_specs=pl.BlockSpec((_ROWS_B, D_OUT), lambda i: (i, 0)),
    out_shape=jax.ShapeDtypeStruct((N, D_OUT), jnp.float32),
)


def kernel(x, edge_index, W, b):
    src_s = edge_index[0].reshape(NS, NCHUNK2, K)
    dst_s = edge_index[1].reshape(NS, NCHUNK2, K)
    sc_degree, sc_aggregate = _sc_kernels()

    zeros_n = jnp.zeros((NPAD,), jnp.float32)
    deg, bsrc, bdst, cnt = sc_degree(src_s, dst_s, zeros_n)
    d0 = jnp.broadcast_to(deg[0, :N, None], (N, D_IN))
    d1 = jnp.broadcast_to(deg[1, :N, None], (N, D_IN))

    y, s = _tc_scale(x, d0, d1)

    bsrc = bsrc.reshape(NC, NS, CCH, K2)
    bdst = bdst.reshape(NC, NS, CCH, K2)
    agg2 = sc_aggregate(y, bsrc, bdst, cnt)
    agg = jnp.concatenate([agg2[0], agg2[1, : N - NHALF]], axis=0)

    b_pad = jnp.broadcast_to(b[None, :], (8, D_OUT))
    return _tc_final(agg, y, s, W, b_pad)


# speedup vs baseline: 1.1705x; 1.1705x over previous
import functools

import jax
import jax.numpy as jnp
from jax import lax
from jax.experimental import pallas as pl
from jax.experimental.pallas import tpu as pltpu
from jax.experimental.pallas import tpu_sc as plsc

N = 10000
E = 320000
D_IN = 128
D_OUT = 512

NC = 2
NS = 16
NW = NC * NS
NPAD = 10240
RPS = NPAD // NS
EPW = E // NW
K = 80
NCHUNK = EPW // K

EPS = E // NS
NCHUNK2 = EPS // K
NHALF = 5120
NACC = 5248
ZPS = NACC // NS
OPS = NHALF // NS
K2 = 80
C = 20080
CCH = C // K2


def _degree_body(src_hbm, dst_hbm, zeros_hbm, deg_hbm, bsrc_hbm, bdst_hbm,
                 cnt_hbm, srcv, dstv, hist, tmp, bsrcv, bdstv, cntv, hist_sh):
    cid = lax.axis_index("c")
    sid = lax.axis_index("s")

    pltpu.sync_copy(zeros_hbm, hist)
    pltpu.sync_copy(src_hbm.at[sid], srcv)
    pltpu.sync_copy(dst_hbm.at[sid], dstv)

    ones16 = jnp.ones((16,), jnp.float32)

    @pl.loop(cid * (NCHUNK2 // 2), (cid + 1) * (NCHUNK2 // 2))
    def _chunk(j):
        for l in range(K // 16):
            idx = dstv[j, pl.ds(l * 16, 16)]
            plsc.addupdate_scatter(hist, [idx], ones16)

    lanes = lax.iota(jnp.int32, 16)

    @pl.loop(0, C // 16)
    def _pf(i):
        bdstv[pl.ds(i * 16, 16)] = NHALF + lanes
        bsrcv[pl.ds(i * 16, 16)] = jnp.zeros((16,), jnp.int32)

    base = cid * NHALF

    @pl.loop(0, NCHUNK2, init_carry=jnp.int32(0))
    def _part(j, off):
        for l in range(K // 16):
            g = pl.ds(l * 16, 16)
            v = dstv[j, g] - base
            ok = (v >= 0) & (v < NHALF)
            plsc.store_compressed(bdstv.at[pl.ds(off, 16)], v, mask=ok)
            plsc.store_compressed(bsrcv.at[pl.ds(off, 16)], srcv[j, g],
                                  mask=ok)
            off = off + jnp.sum(ok.astype(jnp.int32))
        return off

    off = _part
    pltpu.sync_copy(bsrcv, bsrc_hbm.at[cid, sid])
    pltpu.sync_copy(bdstv, bdst_hbm.at[cid, sid])
    cntv[...] = jnp.full((16,), 1, jnp.int32) * off
    pltpu.sync_copy(cntv, cnt_hbm.at[cid, sid])

    pltpu.sync_copy(hist, hist_sh.at[sid])
    plsc.subcore_barrier()

    cols = pl.ds(sid * RPS, RPS)
    pltpu.sync_copy(hist_sh.at[0, cols], hist.at[pl.ds(0, RPS)])
    acc = hist.at[pl.ds(0, RPS)]
    for t in range(1, NS):
        pltpu.sync_copy(hist_sh.at[t, cols], tmp)
        for i in range(RPS // 16):
            g = pl.ds(i * 16, 16)
            acc[g] = acc[g] + tmp[g]

    pltpu.sync_copy(acc, deg_hbm.at[cid, cols])


def _agg_body(y_hbm, src_hbm, dst_hbm, cnt_hbm, agg_hbm, srcv, dstv, cntv,
              rows, sem0, sem1, ssem0, ssem1, agg_sh):
    cid = lax.axis_index("c")
    sid = lax.axis_index("s")

    @pl.loop(0, K2)
    def _z(i):
        for j in range(D_IN // 16):
            rows[i, pl.ds(j * 16, 16)] = jnp.zeros((16,), dtype=jnp.float32)

    buf0 = rows.at[pl.ds(0, K2)]
    buf1 = rows.at[pl.ds(K2, K2)]
    for c in range(ZPS // K2):
        pltpu.sync_copy(buf0, agg_sh.at[pl.ds(sid * ZPS + c * K2, K2)])
    pltpu.sync_copy(
        buf0.at[pl.ds(0, ZPS % K2)],
        agg_sh.at[pl.ds(sid * ZPS + (ZPS // K2) * K2, ZPS % K2)],
    )
    plsc.subcore_barrier()

    pltpu.sync_copy(src_hbm.at[cid, sid], srcv)
    pltpu.sync_copy(dst_hbm.at[cid, sid], dstv)
    pltpu.sync_copy(cnt_hbm.at[cid, sid], cntv)

    count = jnp.max(cntv[...])
    m2 = jnp.maximum(2, 2 * ((count + 2 * K2 - 1) // (2 * K2)))

    def _gather(j, buf, sem):
        return pltpu.async_copy(y_hbm.at[srcv.at[j]], buf, sem)

    def _gwait(j, buf, sem):
        pltpu.make_async_copy(y_hbm.at[srcv.at[j]], buf, sem).wait()

    def _scatter(j, buf, sem):
        return pltpu.async_copy(buf, agg_sh.at[dstv.at[j]], sem, add=True)

    def _swait(j, buf, sem):
        pltpu.make_async_copy(buf, agg_sh.at[dstv.at[j]], sem).wait()

    _gather(0, buf0, sem0)
    _gwait(0, buf0, sem0)
    _scatter(0, buf0, ssem0)
    _gather(1, buf1, sem1)

    @pl.loop(1, m2 // 2)
    def _chunk(g):
        j = 2 * g
        _swait(j - 2, buf0, ssem0)
        _gather(j, buf0, sem0)
        _gwait(j - 1, buf1, sem1)
        _scatter(j - 1, buf1, ssem1)
        _swait(j - 1, buf1, ssem1)
        _gather(j + 1, buf1, sem1)
        _gwait(j, buf0, sem0)
        _scatter(j, buf0, ssem0)

    _gwait(m2 - 1, buf1, sem1)
    _scatter(m2 - 1, buf1, ssem1)
    _swait(m2 - 2, buf0, ssem0)
    _swait(m2 - 1, buf1, ssem1)

    plsc.subcore_barrier()
    pltpu.sync_copy(
        agg_sh.at[pl.ds(sid * OPS, OPS)],
        agg_hbm.at[cid, pl.ds(sid * OPS, OPS)],
    )


@functools.cache
def _sc_kernels():
    mesh = plsc.VectorSubcoreMesh(
        core_axis_name="c", subcore_axis_name="s", num_cores=NC, num_subcores=NS
    )
    sc_degree = functools.partial(
        pl.kernel,
        out_type=(
            jax.ShapeDtypeStruct((NC, NPAD), jnp.float32),
            jax.ShapeDtypeStruct((NC, NS, C), jnp.int32),
            jax.ShapeDtypeStruct((NC, NS, C), jnp.int32),
            jax.ShapeDtypeStruct((NC, NS, 16), jnp.int32),
        ),
        mesh=mesh,
        scratch_types=[
            pltpu.VMEM((NCHUNK2, K), jnp.int32),
            pltpu.VMEM((NCHUNK2, K), jnp.int32),
            pltpu.VMEM((NPAD,), jnp.float32),
            pltpu.VMEM((RPS,), jnp.float32),
            pltpu.VMEM((C,), jnp.int32),
            pltpu.VMEM((C,), jnp.int32),
            pltpu.VMEM((16,), jnp.int32),
            pltpu.VMEM_SHARED((NS, NPAD), jnp.float32),
        ],
        compiler_params=pltpu.CompilerParams(needs_layout_passes=False),
    )(_degree_body)
    sc_aggregate = functools.partial(
        pl.kernel,
        out_type=jax.ShapeDtypeStruct((NC, NHALF, D_IN), jnp.float32),
        mesh=mesh,
        scratch_types=[
            pltpu.VMEM((CCH, K2), jnp.int32),
            pltpu.VMEM((CCH, K2), jnp.int32),
            pltpu.VMEM((16,), jnp.int32),
            pltpu.VMEM((2 * K2, D_IN), jnp.float32),
            pltpu.SemaphoreType.DMA,
            pltpu.SemaphoreType.DMA,
            pltpu.SemaphoreType.DMA,
            pltpu.SemaphoreType.DMA,
            pltpu.VMEM_SHARED((NACC, D_IN), jnp.float32),
        ],
        compiler_params=pltpu.CompilerParams(needs_layout_passes=False),
    )(_agg_body)
    return sc_degree, sc_aggregate


_ROWS_B = 400


def _scale_body(x_ref, d0_ref, d1_ref, y_ref, s_ref):
    s = lax.rsqrt(d0_ref[...] + d1_ref[...] + 1.0)
    y_ref[...] = x_ref[...] * s
    s_ref[...] = s


_tc_scale = pl.pallas_call(
    _scale_body,
    grid=(N // _ROWS_B,),
    in_specs=[pl.BlockSpec((_ROWS_B, D_IN), lambda i: (i, 0))] * 3,
    out_specs=[pl.BlockSpec((_ROWS_B, D_IN), lambda i: (i, 0))] * 2,
    out_shape=[jax.ShapeDtypeStruct((N, D_IN), jnp.float32)] * 2,
)


def _final_body(a_ref, y_ref, s_ref, w_ref, b_ref, o_ref):
    z = (a_ref[...] + y_ref[...]) * s_ref[...]
    acc = jnp.dot(z, w_ref[...], preferred_element_type=jnp.float32)
    o_ref[...] = jnp.maximum(acc + b_ref[...][0:1, :], 0.0)


_tc_final = pl.pallas_call(
    _final_body,
    grid=(N // _ROWS_B,),
    in_specs=[
        pl.BlockSpec((_ROWS_B, D_IN), lambda i: (i, 0)),
        pl.BlockSpec((_ROWS_B, D_IN), lambda i: (i, 0)),
        pl.BlockSpec((_ROWS_B, D_IN), lambda i: (i, 0)),
        pl.BlockSpec((D_IN, D_OUT), lambda i: (0, 0)),
        pl.BlockSpec((8, D_OUT), lambda i: (0, 0)),
    ],
    out_specs=pl.BlockSpec((_ROWS_B, D_OUT), lambda i: (i, 0)),
    out_shape=jax.ShapeDtypeStruct((N, D_OUT), jnp.float32),
)


def kernel(x, edge_index, W, b):
    src_s = edge_index[0].reshape(NS, NCHUNK2, K)
    dst_s = edge_index[1].reshape(NS, NCHUNK2, K)
    sc_degree, sc_aggregate = _sc_kernels()

    zeros_n = jnp.zeros((NPAD,), jnp.float32)
    deg, bsrc, bdst, cnt = sc_degree(src_s, dst_s, zeros_n)
    d0 = jnp.broadcast_to(deg[0, :N, None], (N, D_IN))
    d1 = jnp.broadcast_to(deg[1, :N, None], (N, D_IN))

    y, s = _tc_scale(x, d0, d1)

    bsrc = bsrc.reshape(NC, NS, CCH, K2)
    bdst = bdst.reshape(NC, NS, CCH, K2)
    agg2 = sc_aggregate(y, bsrc, bdst, cnt)
    agg = jnp.concatenate([agg2[0], agg2[1, : N - NHALF]], axis=0)

    b_pad = jnp.broadcast_to(b[None, :], (8, D_OUT))
    return _tc_final(agg, y, s, W, b_pad)


# speedup vs baseline: 1.2364x; 1.0563x over previous
import functools

import jax
import jax.numpy as jnp
from jax import lax
from jax.experimental import pallas as pl
from jax.experimental.pallas import tpu as pltpu
from jax.experimental.pallas import tpu_sc as plsc

N = 10000
E = 320000
D_IN = 128
D_OUT = 512

NC = 2
NS = 16
NW = NC * NS
NPAD = 10240
RPS = NPAD // NS
EPW = E // NW
K = 80
NCHUNK = EPW // K
EPS = E // NS
NCHUNK2 = EPS // K
NHALF = 5120
NACC = 5248
ZPS = NACC // NS
OPS = NHALF // NS


def _degree_body(dst_hbm, zeros_hbm, deg_hbm, dstv, hist, tmp, hist_sh):
    cid = lax.axis_index("c")
    sid = lax.axis_index("s")
    wid = sid * NC + cid

    pltpu.sync_copy(zeros_hbm, hist)
    pltpu.sync_copy(dst_hbm.at[wid], dstv)

    ones16 = jnp.ones((16,), jnp.float32)

    @pl.loop(0, NCHUNK)
    def _chunk(j):
        for l in range(K // 16):
            idx = dstv[j, pl.ds(l * 16, 16)]
            plsc.addupdate_scatter(hist, [idx], ones16)

    pltpu.sync_copy(hist, hist_sh.at[sid])
    plsc.subcore_barrier()

    cols = pl.ds(sid * RPS, RPS)
    pltpu.sync_copy(hist_sh.at[0, cols], hist.at[pl.ds(0, RPS)])
    acc = hist.at[pl.ds(0, RPS)]
    for t in range(1, NS):
        pltpu.sync_copy(hist_sh.at[t, cols], tmp)
        for i in range(RPS // 16):
            g = pl.ds(i * 16, 16)
            acc[g] = acc[g] + tmp[g]

    pltpu.sync_copy(acc, deg_hbm.at[cid, cols])


def _agg_body(y_hbm, src_hbm, dst_hbm, agg_hbm, srcv, dstv, rows, sem0, sem1,
              ssem0, ssem1, agg_sh):
    cid = lax.axis_index("c")
    sid = lax.axis_index("s")

    @pl.loop(0, K)
    def _z(i):
        for j in range(D_IN // 16):
            rows[i, pl.ds(j * 16, 16)] = jnp.zeros((16,), dtype=jnp.float32)

    buf0 = rows.at[pl.ds(0, K)]
    buf1 = rows.at[pl.ds(K, K)]
    for c in range(ZPS // K):
        pltpu.sync_copy(buf0, agg_sh.at[pl.ds(sid * ZPS + c * K, K)])
    pltpu.sync_copy(
        buf0.at[pl.ds(0, ZPS % K)],
        agg_sh.at[pl.ds(sid * ZPS + (ZPS // K) * K, ZPS % K)],
    )
    plsc.subcore_barrier()

    pltpu.sync_copy(src_hbm.at[sid], srcv)
    pltpu.sync_copy(dst_hbm.at[sid], dstv)

    base = cid * NHALF

    @pl.loop(0, NCHUNK2)
    def _t(j):
        for l in range(K // 16):
            v = dstv[j, pl.ds(l * 16, 16)] - base
            ok = (v >= 0) & (v < NHALF)
            dump = NHALF + l * 16 + lax.iota(jnp.int32, 16)
            dstv[j, pl.ds(l * 16, 16)] = jnp.where(ok, v, dump)

    def _gather(j, buf, sem):
        return pltpu.async_copy(y_hbm.at[srcv.at[j]], buf, sem)

    def _gwait(j, buf, sem):
        pltpu.make_async_copy(y_hbm.at[srcv.at[j]], buf, sem).wait()

    def _scatter(j, buf, sem):
        return pltpu.async_copy(buf, agg_sh.at[dstv.at[j]], sem, add=True)

    def _swait(j, buf, sem):
        pltpu.make_async_copy(buf, agg_sh.at[dstv.at[j]], sem).wait()

    _gather(0, buf0, sem0)
    _gwait(0, buf0, sem0)
    _scatter(0, buf0, ssem0)
    _gather(1, buf1, sem1)

    @pl.loop(1, NCHUNK2 // 2)
    def _chunk(g):
        j = 2 * g
        _swait(j - 2, buf0, ssem0)
        _gather(j, buf0, sem0)
        _gwait(j - 1, buf1, sem1)
        _scatter(j - 1, buf1, ssem1)
        _swait(j - 1, buf1, ssem1)
        _gather(j + 1, buf1, sem1)
        _gwait(j, buf0, sem0)
        _scatter(j, buf0, ssem0)

    _gwait(NCHUNK2 - 1, buf1, sem1)
    _scatter(NCHUNK2 - 1, buf1, ssem1)
    _swait(NCHUNK2 - 2, buf0, ssem0)
    _swait(NCHUNK2 - 1, buf1, ssem1)

    plsc.subcore_barrier()
    pltpu.sync_copy(
        agg_sh.at[pl.ds(sid * OPS, OPS)],
        agg_hbm.at[cid, pl.ds(sid * OPS, OPS)],
    )


@functools.cache
def _sc_kernels():
    mesh = plsc.VectorSubcoreMesh(
        core_axis_name="c", subcore_axis_name="s", num_cores=NC, num_subcores=NS
    )
    sc_degree = functools.partial(
        pl.kernel,
        out_type=jax.ShapeDtypeStruct((NC, NPAD), jnp.float32),
        mesh=mesh,
        scratch_types=[
            pltpu.VMEM((NCHUNK, K), jnp.int32),
            pltpu.VMEM((NPAD,), jnp.float32),
            pltpu.VMEM((RPS,), jnp.float32),
            pltpu.VMEM_SHARED((NS, NPAD), jnp.float32),
        ],
        compiler_params=pltpu.CompilerParams(needs_layout_passes=False),
    )(_degree_body)
    sc_aggregate = functools.partial(
        pl.kernel,
        out_type=jax.ShapeDtypeStruct((NC, NHALF, D_IN), jnp.float32),
        mesh=mesh,
        scratch_types=[
            pltpu.VMEM((NCHUNK2, K), jnp.int32),
            pltpu.VMEM((NCHUNK2, K), jnp.int32),
            pltpu.VMEM((2 * K, D_IN), jnp.float32),
            pltpu.SemaphoreType.DMA,
            pltpu.SemaphoreType.DMA,
            pltpu.SemaphoreType.DMA,
            pltpu.SemaphoreType.DMA,
            pltpu.VMEM_SHARED((NACC, D_IN), jnp.float32),
        ],
    )(_agg_body)
    return sc_degree, sc_aggregate


_ROWS_B = 400


def _scale_body(x_ref, d0_ref, d1_ref, y_ref, s_ref):
    s = lax.rsqrt(d0_ref[...] + d1_ref[...] + 1.0)
    y_ref[...] = x_ref[...] * s
    s_ref[...] = s


_tc_scale = pl.pallas_call(
    _scale_body,
    grid=(N // _ROWS_B,),
    in_specs=[pl.BlockSpec((_ROWS_B, D_IN), lambda i: (i, 0))] * 3,
    out_specs=[pl.BlockSpec((_ROWS_B, D_IN), lambda i: (i, 0))] * 2,
    out_shape=[jax.ShapeDtypeStruct((N, D_IN), jnp.float32)] * 2,
)


def _final_body(a_ref, y_ref, s_ref, w_ref, b_ref, o_ref):
    z = (a_ref[...] + y_ref[...]) * s_ref[...]
    acc = jnp.dot(z, w_ref[...], preferred_element_type=jnp.float32)
    o_ref[...] = jnp.maximum(acc + b_ref[...][0:1, :], 0.0)


_tc_final = pl.pallas_call(
    _final_body,
    grid=(N // _ROWS_B,),
    in_specs=[
        pl.BlockSpec((_ROWS_B, D_IN), lambda i: (i, 0)),
        pl.BlockSpec((_ROWS_B, D_IN), lambda i: (i, 0)),
        pl.BlockSpec((_ROWS_B, D_IN), lambda i: (i, 0)),
        pl.BlockSpec((D_IN, D_OUT), lambda i: (0, 0)),
        pl.BlockSpec((8, D_OUT), lambda i: (0, 0)),
    ],
    out_specs=pl.BlockSpec((_ROWS_B, D_OUT), lambda i: (i, 0)),
    out_shape=jax.ShapeDtypeStruct((N, D_OUT), jnp.float32),
)


def kernel(x, edge_index, W, b):
    dst_w = edge_index[1].reshape(NW, NCHUNK, K)
    src_s = edge_index[0].reshape(NS, NCHUNK2, K)
    dst_s = edge_index[1].reshape(NS, NCHUNK2, K)
    sc_degree, sc_aggregate = _sc_kernels()

    zeros_n = jnp.zeros((NPAD,), jnp.float32)
    deg = sc_degree(dst_w, zeros_n)
    d0 = jnp.broadcast_to(deg[0, :N, None], (N, D_IN))
    d1 = jnp.broadcast_to(deg[1, :N, None], (N, D_IN))

    y, s = _tc_scale(x, d0, d1)

    agg2 = sc_aggregate(y, src_s, dst_s)
    agg = jnp.concatenate([agg2[0], agg2[1, : N - NHALF]], axis=0)

    b_pad = jnp.broadcast_to(b[None, :], (8, D_OUT))
    return _tc_final(agg, y, s, W, b_pad)
